# baseline (speedup 1.0000x reference)
import functools

import jax
import jax.numpy as jnp
from jax import lax
from jax.experimental import pallas as pl
from jax.experimental.pallas import tpu as pltpu
from jax.experimental.pallas import tpu_sc as plsc

BATCH = 16384
DIM = 16
LANES = 16
NUM_CORES = 2
NUM_SUBCORES = 16
NUM_WORKERS = NUM_CORES * NUM_SUBCORES
BPW = BATCH // NUM_WORKERS
CHUNKS = BPW // LANES


def _neumf_body(uidx_hbm, iidx_hbm, umf_hbm, imf_hbm, umlp_hbm, imlp_hbm,
                params_hbm, out_hbm,
                uidx_v, iidx_v, umf_v, imf_v, umlp_v, imlp_v, params_v,
                out_v, sem):
    wid = lax.axis_index("s") * NUM_CORES + lax.axis_index("c")
    base = wid * BPW
    pltpu.sync_copy(params_hbm, params_v)
    pltpu.sync_copy(uidx_hbm.at[pl.ds(base, BPW)], uidx_v)
    pltpu.sync_copy(iidx_hbm.at[pl.ds(base, BPW)], iidx_v)
    c1 = pltpu.async_copy(umf_hbm.at[uidx_v], umf_v, sem)
    c2 = pltpu.async_copy(imf_hbm.at[iidx_v], imf_v, sem)
    c3 = pltpu.async_copy(umlp_hbm.at[uidx_v], umlp_v, sem)
    c4 = pltpu.async_copy(imlp_hbm.at[iidx_v], imlp_v, sem)
    c1.wait()
    c2.wait()
    c3.wait()
    c4.wait()

    w_mlp = params_v[pl.ds(0, LANES)]
    w_mf = params_v[pl.ds(DIM, LANES)]
    bias = params_v[pl.ds(2 * DIM, LANES)][0]

    def chunk(c, carry):
        b0 = c * LANES
        rows = b0 + lax.iota(jnp.int32, LANES)
        acc = jnp.full((LANES,), bias, jnp.float32)
        for d in range(DIM):
            col = jnp.full((LANES,), d, jnp.int32)
            gu = plsc.load_gather(umlp_v, [rows, col])
            gi = plsc.load_gather(imlp_v, [rows, col])
            acc = acc + (gu * gi) * w_mlp[d]
            gu2 = plsc.load_gather(umf_v, [rows, col])
            gi2 = plsc.load_gather(imf_v, [rows, col])
            acc = acc + (gu2 * gi2) * w_mf[d]
        out_v[pl.ds(b0, LANES)] = 1.0 / (1.0 + jnp.exp(-acc))
        return carry

    lax.fori_loop(0, CHUNKS, chunk, 0)
    pltpu.sync_copy(out_v, out_hbm.at[pl.ds(base, BPW)])


@jax.jit
def kernel(user_indices, item_indices, emb_user_mf, emb_item_mf,
           emb_user_mlp, emb_item_mlp, affine_w, affine_b):
    params = jnp.concatenate(
        [affine_w[0], affine_b, jnp.zeros((15,), jnp.float32)])
    mesh = plsc.VectorSubcoreMesh(core_axis_name="c", subcore_axis_name="s")
    run = functools.partial(
        pl.kernel,
        mesh=mesh,
        compiler_params=pltpu.CompilerParams(
            needs_layout_passes=False, use_tc_tiling_on_sc=False),
        out_type=jax.ShapeDtypeStruct((BATCH,), jnp.float32),
        scratch_types=[
            pltpu.VMEM((BPW,), jnp.int32),
            pltpu.VMEM((BPW,), jnp.int32),
            pltpu.VMEM((BPW, DIM), jnp.float32),
            pltpu.VMEM((BPW, DIM), jnp.float32),
            pltpu.VMEM((BPW, DIM), jnp.float32),
            pltpu.VMEM((BPW, DIM), jnp.float32),
            pltpu.VMEM((2 * DIM + 16,), jnp.float32),
            pltpu.VMEM((BPW,), jnp.float32),
            pltpu.SemaphoreType.DMA,
        ],
    )(_neumf_body)
    out = run(user_indices.astype(jnp.int32), item_indices.astype(jnp.int32),
              emb_user_mf, emb_item_mf, emb_user_mlp, emb_item_mlp, params)
    return out.reshape(BATCH, 1)
